# baseline (speedup 1.0000x reference)
import jax
import jax.numpy as jnp
from jax import lax
from jax.experimental import pallas as pl
from jax.experimental.pallas import tpu as pltpu
from jax.experimental.pallas import tpu_sc as plsc

_NC = 2
_NS = 16
_NW = _NC * _NS
_L = 16


def _make_kernel(B, X, Y, Z, C, P):
    PPW = P // _NW
    K = 1024
    NCHUNK = PPW // K
    NIDX = 2 * K
    GD = NIDX // 128
    QC = 4 * C

    mesh = plsc.VectorSubcoreMesh(core_axis_name="c", subcore_axis_name="s")

    def body(table, coords, out, coords_v, idx_v, vals_v, out_v, sem):
        cid = lax.axis_index("c")
        sid = lax.axis_index("s")
        wid = cid * _NS + sid
        batch = (wid * PPW) // (P // B)
        b_off = batch * (X * Y * Z)
        base0 = wid * PPW
        iota = lax.iota(jnp.int32, _L)
        iota3 = iota * 3
        iota4 = iota * 4
        zeros = jnp.zeros((_L,), jnp.float32)
        ones = jnp.ones((_L,), jnp.float32)
        cols = [jnp.full((_L,), q, jnp.int32) for q in range(QC)]

        def load_xyz(i0):
            ci = iota3 + (3 * i0)
            x = plsc.load_gather(coords_v, [ci])
            y = plsc.load_gather(coords_v, [ci + 1])
            z = plsc.load_gather(coords_v, [ci + 2])
            return x, y, z

        @pl.loop(0, NCHUNK)
        def chunk_loop(n):
            base = base0 + n * K
            pltpu.sync_copy(coords.at[pl.ds(3 * base, 3 * K)], coords_v)

            @pl.loop(0, K // _L)
            def pass1(jj):
                i0 = jj * _L
                x, y, z = load_xyz(i0)
                xi = x.astype(jnp.int32)
                yi = y.astype(jnp.int32)
                zi = z.astype(jnp.int32)
                x0 = jnp.clip(xi, 0, X - 1)
                x1 = jnp.clip(xi + 1, 0, X - 1)
                y0 = jnp.clip(yi, 0, Y - 1)
                z0 = jnp.clip(zi, 0, Z - 1)
                rbase = y0 * Z + z0 + b_off
                r0 = rbase + x0 * (Y * Z)
                r1 = rbase + x1 * (Y * Z)
                pos0 = iota + i0
                pos1 = pos0 + K
                plsc.store_scatter(idx_v, [pos0 >> 7, pos0 & 127], r0)
                plsc.store_scatter(idx_v, [pos1 >> 7, pos1 & 127], r1)

            cps = [
                pltpu.async_copy(
                    table.at[idx_v.at[g]],
                    vals_v.at[pl.ds(g * 128, 128)],
                    sem,
                )
                for g in range(GD)
            ]
            for cp in cps:
                cp.wait()

            @pl.loop(0, K // _L)
            def pass2(jj):
                i0 = jj * _L
                x, y, z = load_xyz(i0)
                fx = x - x.astype(jnp.int32).astype(jnp.float32)
                fy = y - y.astype(jnp.int32).astype(jnp.float32)
                fz = z - z.astype(jnp.int32).astype(jnp.float32)
                gx = ones - fx
                gy = ones - fy
                gz = ones - fz
                wq = [gy * gz, gy * fz, fy * gz, fy * fz]
                wa = [gx, fx]
                rows0 = iota + i0
                acc = [zeros, zeros, zeros, zeros]
                for a in range(2):
                    r = rows0 + a * K
                    for c in range(C):
                        t = zeros
                        for q in range(4):
                            v = plsc.load_gather(vals_v, [r, cols[q * C + c]])
                            t = t + wq[q] * v
                        acc[c] = acc[c] + wa[a] * t
                oi = iota4 + (4 * i0)
                for c in range(C):
                    plsc.store_scatter(out_v, [oi + c], acc[c])

            pltpu.sync_copy(out_v, out.at[pl.ds(4 * base, 4 * K)])

    grid_kernel = pl.kernel(
        body,
        out_type=jax.ShapeDtypeStruct((P * C,), jnp.float32),
        mesh=mesh,
        scratch_types=[
            pltpu.VMEM((3 * K,), jnp.float32),
            pltpu.VMEM((GD, 128), jnp.int32),
            pltpu.VMEM((NIDX, QC), jnp.float32),
            pltpu.VMEM((4 * K,), jnp.float32),
            pltpu.SemaphoreType.DMA,
        ],
        compiler_params=pltpu.CompilerParams(
            needs_layout_passes=False, use_tc_tiling_on_sc=False),
    )
    return grid_kernel


def kernel(inputs, sample_coords):
    B, X, Y, Z, C = inputs.shape
    d0, d1, d2 = sample_coords.shape[1:4]
    P = B * d0 * d1 * d2
    vz = jnp.concatenate([inputs[:, :, :, 1:, :], inputs[:, :, :, -1:, :]], axis=3)
    vy = jnp.concatenate([inputs[:, :, 1:, :, :], inputs[:, :, -1:, :, :]], axis=2)
    vyz = jnp.concatenate([vz[:, :, 1:, :, :], vz[:, :, -1:, :, :]], axis=2)
    aux = jnp.stack([inputs, vz, vy, vyz], axis=4)
    table = aux.reshape(B * X * Y * Z, 4 * C)
    coords = sample_coords.reshape(P * 3)
    out = _make_kernel(B, X, Y, Z, C, P)(table, coords)
    return out.reshape(B, d0, d1, d2, C)


# speedup vs baseline: 1.2423x; 1.2423x over previous
import jax
import jax.numpy as jnp
from jax import lax
from jax.experimental import pallas as pl
from jax.experimental.pallas import tpu as pltpu
from jax.experimental.pallas import tpu_sc as plsc

_NC = 2
_NS = 16
_NW = _NC * _NS
_L = 16


def _make_kernel(B, X, Y, Z, C, P):
    PPW = P // _NW
    K = 256
    NCHUNK = PPW // K
    NIDX = 2 * K
    GD = NIDX // 128
    QC = 4 * C

    mesh = plsc.VectorSubcoreMesh(core_axis_name="c", subcore_axis_name="s")

    def body(table, coords, out, coords_v, idx_v, vals_v, out_v, sem):
        cid = lax.axis_index("c")
        sid = lax.axis_index("s")
        wid = cid * _NS + sid
        batch = (wid * PPW) // (P // B)
        b_off = batch * (X * Y * Z)
        base0 = wid * PPW
        iota = lax.iota(jnp.int32, _L)
        iota3 = iota * 3
        iota4 = iota * 4
        zeros = jnp.zeros((_L,), jnp.float32)
        ones = jnp.ones((_L,), jnp.float32)

        def load_xyz(i0):
            ci = iota3 + (3 * i0)
            x = plsc.load_gather(coords_v, [ci])
            y = plsc.load_gather(coords_v, [ci + 1])
            z = plsc.load_gather(coords_v, [ci + 2])
            return x, y, z

        def quad_ids(x, y, z):
            xi = x.astype(jnp.int32)
            yi = y.astype(jnp.int32)
            zi = z.astype(jnp.int32)
            x0 = jnp.clip(xi, 0, X - 1)
            x1 = jnp.clip(xi + 1, 0, X - 1)
            y0 = jnp.clip(yi, 0, Y - 1)
            z0 = jnp.clip(zi, 0, Z - 1)
            qbase = y0 * Z + z0 + b_off
            q0 = qbase + x0 * (Y * Z)
            q1 = qbase + x1 * (Y * Z)
            return q0, q1

        @pl.loop(0, NCHUNK)
        def chunk_loop(n):
            base = base0 + n * K
            pltpu.sync_copy(coords.at[pl.ds(3 * base, 3 * K)], coords_v)

            @pl.loop(0, K // _L)
            def pass1(jj):
                i0 = jj * _L
                x, y, z = load_xyz(i0)
                q0, q1 = quad_ids(x, y, z)
                pos0 = iota + i0
                pos1 = pos0 + K
                plsc.store_scatter(idx_v, [pos0 >> 7, pos0 & 127], q0 >> 3)
                plsc.store_scatter(idx_v, [pos1 >> 7, pos1 & 127], q1 >> 3)

            cps = [
                pltpu.async_copy(
                    table.at[idx_v.at[g]],
                    vals_v.at[pl.ds(g * 128, 128)],
                    sem,
                )
                for g in range(GD)
            ]
            for cp in cps:
                cp.wait()

            @pl.loop(0, K // _L)
            def pass2(jj):
                i0 = jj * _L
                x, y, z = load_xyz(i0)
                q0, q1 = quad_ids(x, y, z)
                colb = [(q0 & 7) * QC, (q1 & 7) * QC]
                fx = x - x.astype(jnp.int32).astype(jnp.float32)
                fy = y - y.astype(jnp.int32).astype(jnp.float32)
                fz = z - z.astype(jnp.int32).astype(jnp.float32)
                gx = ones - fx
                gy = ones - fy
                gz = ones - fz
                wq = [gy * gz, gy * fz, fy * gz, fy * fz]
                wa = [gx, fx]
                rows0 = iota + i0
                acc = [zeros, zeros, zeros, zeros]
                for a in range(2):
                    r = rows0 + a * K
                    for c in range(C):
                        t = zeros
                        for q in range(4):
                            v = plsc.load_gather(
                                vals_v, [r, colb[a] + (q * C + c)])
                            t = t + wq[q] * v
                        acc[c] = acc[c] + wa[a] * t
                oi = iota4 + (4 * i0)
                for c in range(C):
                    plsc.store_scatter(out_v, [oi + c], acc[c])

            pltpu.sync_copy(out_v, out.at[pl.ds(4 * base, 4 * K)])

    grid_kernel = pl.kernel(
        body,
        out_type=jax.ShapeDtypeStruct((P * C,), jnp.float32),
        mesh=mesh,
        scratch_types=[
            pltpu.VMEM((3 * K,), jnp.float32),
            pltpu.VMEM((GD, 128), jnp.int32),
            pltpu.VMEM((NIDX, 128), jnp.float32),
            pltpu.VMEM((4 * K,), jnp.float32),
            pltpu.SemaphoreType.DMA,
        ],
        compiler_params=pltpu.CompilerParams(needs_layout_passes=False),
    )
    return grid_kernel


def kernel(inputs, sample_coords):
    B, X, Y, Z, C = inputs.shape
    d0, d1, d2 = sample_coords.shape[1:4]
    P = B * d0 * d1 * d2
    vz = jnp.concatenate([inputs[:, :, :, 1:, :], inputs[:, :, :, -1:, :]], axis=3)
    vy = jnp.concatenate([inputs[:, :, 1:, :, :], inputs[:, :, -1:, :, :]], axis=2)
    vyz = jnp.concatenate([vz[:, :, 1:, :, :], vz[:, :, -1:, :, :]], axis=2)
    aux = jnp.concatenate([inputs, vz, vy, vyz], axis=4)
    table = aux.reshape(B * X * Y * Z // 8, 8 * 4 * C)
    coords = sample_coords.reshape(P * 3)
    out = _make_kernel(B, X, Y, Z, C, P)(table, coords)
    return out.reshape(B, d0, d1, d2, C)


# speedup vs baseline: 4.9538x; 3.9878x over previous
import jax
import jax.numpy as jnp
from jax import lax
from jax.experimental import pallas as pl
from jax.experimental.pallas import tpu as pltpu
from jax.experimental.pallas import tpu_sc as plsc

_NC = 2
_NS = 16
_NW = _NC * _NS
_L = 16


def _make_reformat(B, X, Y, Z, C):
    NLINE = B * X * Y
    LPW = NLINE // _NW
    SHEET = Y
    NSHEET = LPW // SHEET
    mesh = plsc.VectorSubcoreMesh(core_axis_name="c", subcore_axis_name="s")

    def body(vol2d, table, sheet_v, out_v, sem):
        cid = lax.axis_index("c")
        sid = lax.axis_index("s")
        wid = cid * _NS + sid
        line0 = wid * LPW
        j = lax.iota(jnp.int32, _L)
        q = j >> 2
        ch = j & 3
        dy = q >> 1
        c_clamp = q & 1
        c_hi = c_clamp + ch * Z
        c_lo = c_hi + dy * (C * Z)

        @pl.loop(0, NSHEET)
        def sheet_loop(s):
            sheet_line0 = line0 + s * SHEET
            pltpu.sync_copy(
                vol2d.at[pl.ds(sheet_line0 * C, SHEET * C)], sheet_v)

            @pl.loop(0, SHEET)
            def line_loop(y):
                cy = jnp.where(y < SHEET - 1, c_lo, c_hi)
                cyz = cy - c_clamp

                @pl.loop(0, Z)
                def z_loop(z):
                    cz = jnp.where(z < Z - 1, cy, cyz)
                    idx = cz + (y * (C * Z) + z)
                    v = plsc.load_gather(sheet_v, [idx >> 7, idx & (Z - 1)])
                    out_v[z >> 3, pl.ds((z & 7) * 16, 16)] = v

                pltpu.sync_copy(
                    out_v, table.at[pl.ds((sheet_line0 + y) * (Z // 8), Z // 8)])

    return pl.kernel(
        body,
        out_type=jax.ShapeDtypeStruct((B * X * Y * Z // 8, 8 * 4 * C), jnp.float32),
        mesh=mesh,
        scratch_types=[
            pltpu.VMEM((SHEET * C, Z), jnp.float32),
            pltpu.VMEM((Z // 8, 8 * 4 * C), jnp.float32),
            pltpu.SemaphoreType.DMA,
        ],
        compiler_params=pltpu.CompilerParams(needs_layout_passes=False),
    )


def _make_kernel(B, X, Y, Z, C, P, NL):
    PPW = P // _NW
    K = 384
    NCHUNK = PPW // K
    NIDX = 2 * K
    GD = NIDX // 128
    QC = 4 * C

    mesh = plsc.VectorSubcoreMesh(core_axis_name="c", subcore_axis_name="s")

    def body(table, coords, out, coords_v, idx_v, vals_v, out_v, sem):
        cid = lax.axis_index("c")
        sid = lax.axis_index("s")
        wid = cid * _NS + sid
        batch = (wid * PPW) // (P // B)
        b_off = batch * (X * Y * Z)
        base0 = wid * PPW
        iota = lax.iota(jnp.int32, _L)
        zeros = jnp.zeros((_L,), jnp.float32)
        ones = jnp.ones((_L,), jnp.float32)

        def load_xyz(i0):
            x = coords_v[pl.ds(i0, _L)]
            y = coords_v[pl.ds(K + i0, _L)]
            z = coords_v[pl.ds(2 * K + i0, _L)]
            return x, y, z

        def quad_ids(x, y, z):
            xi = x.astype(jnp.int32)
            yi = y.astype(jnp.int32)
            zi = z.astype(jnp.int32)
            x0 = jnp.clip(xi, 0, X - 1)
            x1 = jnp.clip(xi + 1, 0, X - 1)
            y0 = jnp.clip(yi, 0, Y - 1)
            z0 = jnp.clip(zi, 0, Z - 1)
            qbase = y0 * Z + z0 + b_off
            q0 = qbase + x0 * (Y * Z)
            q1 = qbase + x1 * (Y * Z)
            return q0, q1

        @pl.loop(0, NCHUNK)
        def chunk_loop(n):
            p0 = base0 + n * K
            plane = p0 // NL
            s = p0 - plane * NL
            cbase = plane * (3 * NL) + s
            pltpu.sync_copy(coords.at[pl.ds(cbase, K)], coords_v.at[pl.ds(0, K)])
            pltpu.sync_copy(coords.at[pl.ds(cbase + NL, K)],
                            coords_v.at[pl.ds(K, K)])
            pltpu.sync_copy(coords.at[pl.ds(cbase + 2 * NL, K)],
                            coords_v.at[pl.ds(2 * K, K)])

            @pl.loop(0, K // _L)
            def pass1(jj):
                i0 = jj * _L
                x, y, z = load_xyz(i0)
                q0, q1 = quad_ids(x, y, z)
                pos0 = iota + i0
                pos1 = pos0 + K
                plsc.store_scatter(idx_v, [pos0 >> 7, pos0 & 127], q0 >> 3)
                plsc.store_scatter(idx_v, [pos1 >> 7, pos1 & 127], q1 >> 3)

            cps = [
                pltpu.async_copy(
                    table.at[idx_v.at[g]],
                    vals_v.at[pl.ds(g * 128, 128)],
                    sem,
                )
                for g in range(GD)
            ]
            for cp in cps:
                cp.wait()

            @pl.loop(0, K // _L)
            def pass2(jj):
                i0 = jj * _L
                x, y, z = load_xyz(i0)
                q0, q1 = quad_ids(x, y, z)
                colb = [(q0 & 7) * QC, (q1 & 7) * QC]
                fx = x - x.astype(jnp.int32).astype(jnp.float32)
                fy = y - y.astype(jnp.int32).astype(jnp.float32)
                fz = z - z.astype(jnp.int32).astype(jnp.float32)
                gx = ones - fx
                gy = ones - fy
                gz = ones - fz
                wq = [gy * gz, gy * fz, fy * gz, fy * fz]
                wa = [gx, fx]
                rows0 = iota + i0
                acc = [zeros, zeros, zeros, zeros]
                for a in range(2):
                    r = rows0 + a * K
                    for c in range(C):
                        t = zeros
                        for q in range(4):
                            v = plsc.load_gather(
                                vals_v, [r, colb[a] + (q * C + c)])
                            t = t + wq[q] * v
                        acc[c] = acc[c] + wa[a] * t
                line = i0 // 96
                within = i0 - line * 96
                for c in range(C):
                    out_v[pl.ds(line * (96 * C) + c * 96 + within, _L)] = acc[c]

            pltpu.sync_copy(out_v, out.at[pl.ds(4 * p0, 4 * K)])

    return pl.kernel(
        body,
        out_type=jax.ShapeDtypeStruct((P * C,), jnp.float32),
        mesh=mesh,
        scratch_types=[
            pltpu.VMEM((3 * K,), jnp.float32),
            pltpu.VMEM((GD, 128), jnp.int32),
            pltpu.VMEM((NIDX, 128), jnp.float32),
            pltpu.VMEM((4 * K,), jnp.float32),
            pltpu.SemaphoreType.DMA,
        ],
        compiler_params=pltpu.CompilerParams(needs_layout_passes=False),
    )


def kernel(inputs, sample_coords):
    B, X, Y, Z, C = inputs.shape
    d0, d1, d2 = sample_coords.shape[1:4]
    P = B * d0 * d1 * d2
    NL = d1 * d2
    vol2d = inputs.transpose(0, 1, 2, 4, 3).reshape(B * X * Y * C, Z)
    table = _make_reformat(B, X, Y, Z, C)(vol2d)
    coords = sample_coords.transpose(0, 1, 4, 2, 3).reshape(P * 3)
    out = _make_kernel(B, X, Y, Z, C, P, NL)(table, coords)
    return out.reshape(B, d0, d1, C, d2).transpose(0, 1, 2, 4, 3)


# speedup vs baseline: 5.7363x; 1.1580x over previous
import jax
import jax.numpy as jnp
from jax import lax
from jax.experimental import pallas as pl
from jax.experimental.pallas import tpu as pltpu
from jax.experimental.pallas import tpu_sc as plsc

_NC = 2
_NS = 16
_NW = _NC * _NS
_L = 16


def _make_reformat(B, X, Y, Z, C):
    NLINE = B * X * Y
    LPW = NLINE // _NW
    SHEET = Y
    NSHEET = LPW // SHEET
    mesh = plsc.VectorSubcoreMesh(core_axis_name="c", subcore_axis_name="s")

    def body(vol2d, table, sheet_v, out_v, sem):
        cid = lax.axis_index("c")
        sid = lax.axis_index("s")
        wid = cid * _NS + sid
        line0 = wid * LPW
        j = lax.iota(jnp.int32, _L)
        q = j >> 2
        ch = j & 3
        dy = q >> 1
        c_clamp = q & 1
        c_hi = c_clamp + ch * Z
        c_lo = c_hi + dy * (C * Z)

        @pl.loop(0, NSHEET)
        def sheet_loop(s):
            sheet_line0 = line0 + s * SHEET
            pltpu.sync_copy(
                vol2d.at[pl.ds(sheet_line0 * C, SHEET * C)], sheet_v)

            @pl.loop(0, SHEET)
            def line_loop(y):
                cy = jnp.where(y < SHEET - 1, c_lo, c_hi) + y * (C * Z)
                cyz = cy - c_clamp

                @pl.loop(0, Z, unroll=8)
                def z_loop(z):
                    idx = jnp.where(z < Z - 1, cy, cyz) + z
                    v = plsc.load_gather(sheet_v, [idx >> 7, idx & (Z - 1)])
                    out_v[z >> 3, pl.ds((z & 7) * 16, 16)] = v

                pltpu.sync_copy(
                    out_v, table.at[pl.ds((sheet_line0 + y) * (Z // 8), Z // 8)])

    return pl.kernel(
        body,
        out_type=jax.ShapeDtypeStruct((B * X * Y * Z // 8, 8 * 4 * C), jnp.float32),
        mesh=mesh,
        scratch_types=[
            pltpu.VMEM((SHEET * C, Z), jnp.float32),
            pltpu.VMEM((Z // 8, 8 * 4 * C), jnp.float32),
            pltpu.SemaphoreType.DMA,
        ],
        compiler_params=pltpu.CompilerParams(needs_layout_passes=False),
    )


def _make_kernel(B, X, Y, Z, C, P, NL):
    PPW = P // _NW
    K = 192
    NCHUNK = PPW // K
    NIDX = 2 * K
    GD = NIDX // 128
    QC = 4 * C

    mesh = plsc.VectorSubcoreMesh(core_axis_name="c", subcore_axis_name="s")

    def body(table, coords, out,
             coords_v0, coords_v1, idx_v0, idx_v1, vals_v0, vals_v1,
             out_v0, out_v1, sem0, sem1):
        coords_b = [coords_v0, coords_v1]
        idx_b = [idx_v0, idx_v1]
        vals_b = [vals_v0, vals_v1]
        out_b = [out_v0, out_v1]
        sem_b = [sem0, sem1]
        cid = lax.axis_index("c")
        sid = lax.axis_index("s")
        wid = cid * _NS + sid
        batch = (wid * PPW) // (P // B)
        b_off = batch * (X * Y * Z)
        base0 = wid * PPW
        iota = lax.iota(jnp.int32, _L)
        zeros = jnp.zeros((_L,), jnp.float32)
        ones = jnp.ones((_L,), jnp.float32)

        def load_xyz(coords_v, i0):
            x = coords_v[pl.ds(i0, _L)]
            y = coords_v[pl.ds(K + i0, _L)]
            z = coords_v[pl.ds(2 * K + i0, _L)]
            return x, y, z

        def quad_ids(x, y, z):
            xi = x.astype(jnp.int32)
            yi = y.astype(jnp.int32)
            zi = z.astype(jnp.int32)
            x0 = jnp.clip(xi, 0, X - 1)
            x1 = jnp.clip(xi + 1, 0, X - 1)
            y0 = jnp.clip(yi, 0, Y - 1)
            z0 = jnp.clip(zi, 0, Z - 1)
            qbase = y0 * Z + z0 + b_off
            q0 = qbase + x0 * (Y * Z)
            q1 = qbase + x1 * (Y * Z)
            return q0, q1

        def stage1(n, b):
            coords_v, idx_v, vals_v, sem = coords_b[b], idx_b[b], vals_b[b], sem_b[b]
            p0 = base0 + n * K
            plane = p0 // NL
            s = p0 - plane * NL
            cbase = plane * (3 * NL) + s
            pltpu.sync_copy(coords.at[pl.ds(cbase, K)], coords_v.at[pl.ds(0, K)])
            pltpu.sync_copy(coords.at[pl.ds(cbase + NL, K)],
                            coords_v.at[pl.ds(K, K)])
            pltpu.sync_copy(coords.at[pl.ds(cbase + 2 * NL, K)],
                            coords_v.at[pl.ds(2 * K, K)])

            @pl.loop(0, K // _L)
            def pass1(jj):
                i0 = jj * _L
                x, y, z = load_xyz(coords_v, i0)
                q0, q1 = quad_ids(x, y, z)
                pos0 = iota + i0
                pos1 = pos0 + K
                plsc.store_scatter(idx_v, [pos0 >> 7, pos0 & 127], q0 >> 3)
                plsc.store_scatter(idx_v, [pos1 >> 7, pos1 & 127], q1 >> 3)

            for g in range(GD):
                pltpu.async_copy(
                    table.at[idx_v.at[g]],
                    vals_v.at[pl.ds(g * 128, 128)],
                    sem,
                )

        def stage2(n, b):
            coords_v, idx_v, vals_v, sem = coords_b[b], idx_b[b], vals_b[b], sem_b[b]
            out_v = out_b[b]
            p0 = base0 + n * K
            for g in range(GD):
                pltpu.make_async_copy(
                    table.at[idx_v.at[g]],
                    vals_v.at[pl.ds(g * 128, 128)],
                    sem,
                ).wait()

            @pl.loop(0, K // _L)
            def pass2(jj):
                i0 = jj * _L
                x, y, z = load_xyz(coords_v, i0)
                q0, q1 = quad_ids(x, y, z)
                colb = [(q0 & 7) * QC, (q1 & 7) * QC]
                fx = x - x.astype(jnp.int32).astype(jnp.float32)
                fy = y - y.astype(jnp.int32).astype(jnp.float32)
                fz = z - z.astype(jnp.int32).astype(jnp.float32)
                gx = ones - fx
                gy = ones - fy
                gz = ones - fz
                wq = [gy * gz, gy * fz, fy * gz, fy * fz]
                wa = [gx, fx]
                rows0 = iota + i0
                acc = [zeros, zeros, zeros, zeros]
                for a in range(2):
                    r = rows0 + a * K
                    for c in range(C):
                        t = zeros
                        for q in range(4):
                            v = plsc.load_gather(
                                vals_v, [r, colb[a] + (q * C + c)])
                            t = t + wq[q] * v
                        acc[c] = acc[c] + wa[a] * t
                line = i0 // 96
                within = i0 - line * 96
                for c in range(C):
                    out_v[pl.ds(line * (96 * C) + c * 96 + within, _L)] = acc[c]

            pltpu.sync_copy(out_v, out.at[pl.ds(4 * p0, 4 * K)])

        stage1(jnp.int32(0), 0)
        stage1(jnp.int32(1), 1)

        @pl.loop(0, NCHUNK // 2 - 1)
        def chunk_pair(m):
            n0 = 2 * m
            stage2(n0, 0)
            stage1(n0 + 2, 0)
            stage2(n0 + 1, 1)
            stage1(n0 + 3, 1)

        stage2(jnp.int32(NCHUNK - 2), 0)
        stage2(jnp.int32(NCHUNK - 1), 1)

    return pl.kernel(
        body,
        out_type=jax.ShapeDtypeStruct((P * C,), jnp.float32),
        mesh=mesh,
        scratch_types=[
            pltpu.VMEM((3 * K,), jnp.float32),
            pltpu.VMEM((3 * K,), jnp.float32),
            pltpu.VMEM((GD, 128), jnp.int32),
            pltpu.VMEM((GD, 128), jnp.int32),
            pltpu.VMEM((NIDX, 128), jnp.float32),
            pltpu.VMEM((NIDX, 128), jnp.float32),
            pltpu.VMEM((4 * K,), jnp.float32),
            pltpu.VMEM((4 * K,), jnp.float32),
            pltpu.SemaphoreType.DMA,
            pltpu.SemaphoreType.DMA,
        ],
        compiler_params=pltpu.CompilerParams(needs_layout_passes=False),
    )


def kernel(inputs, sample_coords):
    B, X, Y, Z, C = inputs.shape
    d0, d1, d2 = sample_coords.shape[1:4]
    P = B * d0 * d1 * d2
    NL = d1 * d2
    vol2d = inputs.transpose(0, 1, 2, 4, 3).reshape(B * X * Y * C, Z)
    table = _make_reformat(B, X, Y, Z, C)(vol2d)
    coords = sample_coords.transpose(0, 1, 4, 2, 3).reshape(P * 3)
    out = _make_kernel(B, X, Y, Z, C, P, NL)(table, coords)
    return out.reshape(B, d0, d1, C, d2).transpose(0, 1, 2, 4, 3)
